# Initial kernel scaffold; baseline (speedup 1.0000x reference)
import jax
import jax.numpy as jnp
from jax.experimental import pallas as pl

_EPS = 1e-05
_OUT = 512
_KSZ = 8
_ROWS_PER_BLOCK = 256


def _lrp_pool_body(a_ref, r_ref, o_ref):
    x = a_ref[...]
    rr = r_ref[...]
    rb = x.shape[0]
    x3 = x.reshape(rb, _OUT, _KSZ)
    z = jnp.mean(x3, axis=-1) + _EPS
    c = rr / (z * _KSZ)
    o_ref[...] = (x3 * c[:, :, None]).reshape(rb, _OUT * _KSZ)


def kernel(a, r):
    B, C, L = a.shape
    R = B * C
    a2 = a.reshape(R, L)
    r2 = r.reshape(R, _OUT)
    rb = _ROWS_PER_BLOCK
    out = pl.pallas_call(
        _lrp_pool_body,
        grid=(R // rb,),
        in_specs=[
            pl.BlockSpec((rb, L), lambda i: (i, 0)),
            pl.BlockSpec((rb, _OUT), lambda i: (i, 0)),
        ],
        out_specs=pl.BlockSpec((rb, L), lambda i: (i, 0)),
        out_shape=jax.ShapeDtypeStruct((R, L), a.dtype),
    )(a2, r2)
    return out.reshape(B, C, L)


# speedup vs baseline: 1.2423x; 1.2423x over previous
import jax
import jax.numpy as jnp
from jax.experimental import pallas as pl
from jax.experimental.pallas import tpu as pltpu

_EPS = 1e-05
_OUT = 512
_KSZ = 8
_ROWS_PER_BLOCK = 256


def _lrp_pool_body(a_ref, r_ref, o_ref):
    x = a_ref[...]
    rr = r_ref[...]
    lane = jax.lax.broadcasted_iota(jnp.int32, x.shape, 1)
    acc = x
    n = x.shape[1]
    for s in (4, 2, 1):
        fwd = pltpu.roll(acc, n - s, axis=1)
        bwd = pltpu.roll(acc, s, axis=1)
        acc = acc + jnp.where((lane & s) == 0, fwd, bwd)
    z_full = acc * (1.0 / _KSZ) + _EPS
    rb = x.shape[0]
    idx = jax.lax.broadcasted_iota(jnp.int32, (rb, 128 * _KSZ), 1) // _KSZ
    parts = [
        jnp.take_along_axis(rr[:, q * 128:(q + 1) * 128], idx, axis=1)
        for q in range(_OUT // 128)
    ]
    r_full = jnp.concatenate(parts, axis=1)
    c_full = (r_full / z_full) * (1.0 / _KSZ)
    o_ref[...] = x * c_full


def kernel(a, r):
    B, C, L = a.shape
    R = B * C
    a2 = a.reshape(R, L)
    r2 = r.reshape(R, _OUT)
    rb = _ROWS_PER_BLOCK
    out = pl.pallas_call(
        _lrp_pool_body,
        grid=(R // rb,),
        in_specs=[
            pl.BlockSpec((rb, L), lambda i: (i, 0)),
            pl.BlockSpec((rb, _OUT), lambda i: (i, 0)),
        ],
        out_specs=pl.BlockSpec((rb, L), lambda i: (i, 0)),
        out_shape=jax.ShapeDtypeStruct((R, L), a.dtype),
    )(a2, r2)
    return out.reshape(B, C, L)
